# baseline (speedup 1.0000x reference)
import jax
import jax.numpy as jnp
from jax import lax
from jax.experimental import pallas as pl
from jax.experimental.pallas import tpu as pltpu
from jax.experimental.pallas import tpu_sc as plsc

V = 100000
D = 300
B = 4096
H = 50

NC = 2
NS = 16
L = 16
NW = NC * NS
ROWS = 2 * H
SCALE = 1.0 / H

CHUNK_OFFS = tuple(c * L for c in range(D // L)) + (D - L,)


def _build(batch):
    bags_per_w = batch // NW
    pairs_per_w = bags_per_w // 2

    def acc_pair(rows_ref, out_ref, bag0):
        zero = jnp.zeros((L,), jnp.float32)
        for half in range(2):
            def bodyf(r, accs, _half=half):
                row = _half * H + r
                return tuple(a + rows_ref[row, pl.ds(off, L)]
                             for a, off in zip(accs, CHUNK_OFFS))
            accs = lax.fori_loop(0, H, bodyf, tuple(zero for _ in CHUNK_OFFS))
            for a, off in zip(accs, CHUNK_OFFS):
                out_ref[bag0 + half, pl.ds(off, L)] = a * SCALE

    def body(table_hbm, x2_hbm, out_hbm, idx_a, idx_b,
             buf_a, buf_b, out_v, sem_a, sem_b, isem_a, isem_b):
        wid = lax.axis_index("s") * NC + lax.axis_index("c")
        jbase = wid * pairs_per_w
        last = pairs_per_w - 1

        def istart(idx, isem, j):
            jg = jbase + jnp.minimum(j, last)
            pltpu.make_async_copy(x2_hbm.at[jg], idx, isem).start()

        def iwait(idx, isem):
            pltpu.make_async_copy(x2_hbm.at[jbase], idx, isem).wait()

        def row_starts(ivec, r0, lanes, buf, sem):
            for t in lanes:
                i = ivec[t]
                pltpu.make_async_copy(table_hbm.at[i],
                                      buf.at[r0 + t], sem).start()

        def gstart(idx, buf, sem):
            for r0 in (0, 16, 32, 48, 64, 80):
                ivec = idx[pl.ds(r0, L)]
                row_starts(ivec, r0, range(L), buf, sem)
            ivec = idx[pl.ds(ROWS - L, L)]
            row_starts(ivec, ROWS - L, (12, 13, 14, 15), buf, sem)
            for t in range(4):
                pltpu.make_async_copy(table_hbm.at[0],
                                      buf.at[ROWS + t], sem).start()

        def gwait(buf, sem):
            for _ in range((ROWS + 4) // 8):
                pltpu.make_async_copy(table_hbm.at[pl.ds(0, 8)],
                                      buf.at[pl.ds(0, 8)], sem).wait()

        istart(idx_a, isem_a, 0)
        istart(idx_b, isem_b, 1)
        iwait(idx_a, isem_a)
        gstart(idx_a, buf_a, sem_a)
        iwait(idx_b, isem_b)
        gstart(idx_b, buf_b, sem_b)

        def outer(g, carry):
            gwait(buf_a, sem_a)
            istart(idx_a, isem_a, 2 * g + 2)
            acc_pair(buf_a, out_v, 4 * g)
            iwait(idx_a, isem_a)
            gstart(idx_a, buf_a, sem_a)
            gwait(buf_b, sem_b)
            istart(idx_b, isem_b, 2 * g + 3)
            acc_pair(buf_b, out_v, 4 * g + 2)
            iwait(idx_b, isem_b)
            gstart(idx_b, buf_b, sem_b)
            return carry

        lax.fori_loop(0, pairs_per_w // 2, outer, 0)
        gwait(buf_a, sem_a)
        gwait(buf_b, sem_b)
        pltpu.sync_copy(out_v, out_hbm.at[pl.ds(wid * bags_per_w, bags_per_w)])

    out_type = jax.ShapeDtypeStruct((batch, D), jnp.float32)
    scratch_types = [
        pltpu.VMEM((ROWS,), jnp.int32),
        pltpu.VMEM((ROWS,), jnp.int32),
        pltpu.VMEM((ROWS + 4, D), jnp.float32),
        pltpu.VMEM((ROWS + 4, D), jnp.float32),
        pltpu.VMEM((bags_per_w, D), jnp.float32),
        pltpu.SemaphoreType.DMA,
        pltpu.SemaphoreType.DMA,
        pltpu.SemaphoreType.DMA,
        pltpu.SemaphoreType.DMA,
    ]
    return body, out_type, scratch_types


_body, _out_type, _scratch_types = _build(B)
_embed_mean = pl.kernel(
    _body,
    out_type=_out_type,
    mesh=plsc.VectorSubcoreMesh(core_axis_name="c", subcore_axis_name="s"),
    scratch_types=_scratch_types,
    compiler_params=pltpu.CompilerParams(use_tc_tiling_on_sc=True,
                                         needs_layout_passes=False),
)


def kernel(x, table):
    x2 = x.reshape(B // 2, ROWS)
    return _embed_mean(table, x2)


# speedup vs baseline: 1.6263x; 1.6263x over previous
import jax
import jax.numpy as jnp
from jax import lax
from jax.experimental import pallas as pl
from jax.experimental.pallas import tpu as pltpu
from jax.experimental.pallas import tpu_sc as plsc

V = 100000
D = 300
B = 4096
H = 50

NC = 2
NS = 16
L = 16
NW = NC * NS
ROWS = 2 * H
SCALE = 1.0 / H

CHUNK_OFFS = tuple(c * L for c in range(D // L)) + (D - L,)


def _build(batch):
    bags_per_w = batch // NW
    pairs_per_w = bags_per_w // 2

    def acc_pair(rows_ref, out_ref, bag0):
        zero = jnp.zeros((L,), jnp.float32)
        for half in range(2):
            def bodyf(r, accs, _half=half):
                row = _half * H + r
                return tuple(a + rows_ref[row, pl.ds(off, L)]
                             for a, off in zip(accs, CHUNK_OFFS))
            accs = lax.fori_loop(0, H, bodyf, tuple(zero for _ in CHUNK_OFFS))
            for a, off in zip(accs, CHUNK_OFFS):
                out_ref[bag0 + half, pl.ds(off, L)] = a * SCALE

    def body(table_hbm, x2_hbm, out_hbm, idx_a, idx_b,
             buf_a, buf_b, out_v, sem_a, sem_b, isem_a, isem_b):
        wid = lax.axis_index("s") * NC + lax.axis_index("c")
        jbase = wid * pairs_per_w
        last = pairs_per_w - 1

        def istart(idx, isem, j):
            jg = jbase + jnp.minimum(j, last)
            pltpu.make_async_copy(x2_hbm.at[jg], idx, isem).start()

        def iwait(idx, isem):
            pltpu.make_async_copy(x2_hbm.at[jbase], idx, isem).wait()

        def row_starts(ivec, r0, lanes, buf, sem):
            for t in lanes:
                i = ivec[t]
                pltpu.make_async_copy(table_hbm.at[i],
                                      buf.at[r0 + t], sem).start()

        def gstart(idx, buf, sem):
            for r0 in (0, 16, 32, 48, 64, 80):
                ivec = idx[pl.ds(r0, L)]
                row_starts(ivec, r0, range(L), buf, sem)
            ivec = idx[pl.ds(ROWS - L, L)]
            row_starts(ivec, ROWS - L, (12, 13, 14, 15), buf, sem)

        def gwait(buf, sem):
            for _ in range(ROWS):
                pltpu.make_async_copy(table_hbm.at[0], buf.at[0], sem).wait()

        istart(idx_a, isem_a, 0)
        istart(idx_b, isem_b, 1)
        iwait(idx_a, isem_a)
        gstart(idx_a, buf_a, sem_a)
        iwait(idx_b, isem_b)
        gstart(idx_b, buf_b, sem_b)

        def outer(g, carry):
            gwait(buf_a, sem_a)
            istart(idx_a, isem_a, 2 * g + 2)
            acc_pair(buf_a, out_v, 4 * g)
            iwait(idx_a, isem_a)
            gstart(idx_a, buf_a, sem_a)
            gwait(buf_b, sem_b)
            istart(idx_b, isem_b, 2 * g + 3)
            acc_pair(buf_b, out_v, 4 * g + 2)
            iwait(idx_b, isem_b)
            gstart(idx_b, buf_b, sem_b)
            return carry

        lax.fori_loop(0, pairs_per_w // 2, outer, 0)
        gwait(buf_a, sem_a)
        gwait(buf_b, sem_b)
        pltpu.sync_copy(out_v, out_hbm.at[pl.ds(wid * bags_per_w, bags_per_w)])

    out_type = jax.ShapeDtypeStruct((batch, D), jnp.float32)
    scratch_types = [
        pltpu.VMEM((ROWS,), jnp.int32),
        pltpu.VMEM((ROWS,), jnp.int32),
        pltpu.VMEM((ROWS, D), jnp.float32),
        pltpu.VMEM((ROWS, D), jnp.float32),
        pltpu.VMEM((bags_per_w, D), jnp.float32),
        pltpu.SemaphoreType.DMA,
        pltpu.SemaphoreType.DMA,
        pltpu.SemaphoreType.DMA,
        pltpu.SemaphoreType.DMA,
    ]
    return body, out_type, scratch_types


_body, _out_type, _scratch_types = _build(B)
_embed_mean = pl.kernel(
    _body,
    out_type=_out_type,
    mesh=plsc.VectorSubcoreMesh(core_axis_name="c", subcore_axis_name="s"),
    scratch_types=_scratch_types,
    compiler_params=pltpu.CompilerParams(use_tc_tiling_on_sc=True,
                                         needs_layout_passes=False),
)


def kernel(x, table):
    x2 = x.reshape(B // 2, ROWS)
    return _embed_mean(table, x2)
